# baseline (speedup 1.0000x reference)
import jax
import jax.numpy as jnp
from jax.experimental import pallas as pl
from jax.experimental.pallas import tpu as pltpu

_NUM_AGENT = 2048
_EMB = 256
_IN = 2 * _EMB
_DEPTH = 2
_M = 512


def _mm(x, y):
    return jax.lax.dot_general(
        x, y, dimension_numbers=(((1,), (0,)), ((), ())),
        preferred_element_type=jnp.float32)


def _dhgn_block(h0_ref, a_ref, adj_ref, wagg_ref, bagg_ref, wfcra_ref,
                bfcra_ref, out_ref, h_scr):
    k = pl.program_id(0)
    i = pl.program_id(1)
    h_prev = jnp.where(k == 0, h0_ref[...], h_scr[pl.ds(i * _M, _M), :])
    bagg = jnp.where(k == 0, bagg_ref[0], bagg_ref[1])
    bfcra = jnp.where(k == 0, bfcra_ref[0], bfcra_ref[1])
    adj = adj_ref[0]
    deg = jnp.clip(jnp.sum(adj, axis=-1, keepdims=True), 1e-6, None)
    agg = _mm(adj, a_ref[0]) / deg
    emb = jnp.maximum(_mm(agg, wagg_ref[0]) + bagg[None, :], 0.0)
    acc = _mm(emb, wfcra_ref[0, :_EMB]) + _mm(h_prev, wfcra_ref[0, _EMB:])
    h = jnp.maximum(acc + bfcra[None, :], 0.0)
    h_scr[pl.ds(i * _M, _M), :] = h
    out_ref[...] = h


@jax.jit
def kernel(h0, a, adjacent_mat, W_agg, b_agg, W_fcra, b_fcra):
    grid = (_DEPTH, _NUM_AGENT // _M)
    return pl.pallas_call(
        _dhgn_block,
        grid=grid,
        in_specs=[
            pl.BlockSpec((_M, _EMB), lambda k, i: (i, 0)),
            pl.BlockSpec((1, _NUM_AGENT, _IN), lambda k, i: (k, 0, 0)),
            pl.BlockSpec((1, _M, _NUM_AGENT), lambda k, i: (k, i, 0)),
            pl.BlockSpec((1, _IN, _EMB), lambda k, i: (k, 0, 0)),
            pl.BlockSpec((_DEPTH, _EMB), lambda k, i: (0, 0)),
            pl.BlockSpec((1, _IN, _EMB), lambda k, i: (k, 0, 0)),
            pl.BlockSpec((_DEPTH, _EMB), lambda k, i: (0, 0)),
        ],
        out_specs=pl.BlockSpec((_M, _EMB), lambda k, i: (i, 0)),
        out_shape=jax.ShapeDtypeStruct((_NUM_AGENT, _EMB), jnp.float32),
        scratch_shapes=[pltpu.VMEM((_NUM_AGENT, _EMB), jnp.float32)],
    )(h0, a, adjacent_mat, W_agg, b_agg, W_fcra, b_fcra)


# speedup vs baseline: 1.0865x; 1.0865x over previous
import jax
import jax.numpy as jnp
from jax.experimental import pallas as pl

_NUM_AGENT = 2048
_EMB = 256
_IN = 2 * _EMB
_DEPTH = 2
_M = 512
_HALF = _NUM_AGENT // 2


def _mm(x, y):
    return jax.lax.dot_general(
        x, y, dimension_numbers=(((1,), (0,)), ((), ())),
        preferred_element_type=jnp.float32)


def _dhgn_block(h0_ref, a_ref, adjL_ref, adjR_ref, wagg_ref, bagg_ref,
                wfcra_ref, bfcra_ref, out_ref):
    h = h0_ref[...]
    for k in range(_DEPTH):
        adjL = adjL_ref[k]
        adjR = adjR_ref[k]
        deg = jnp.sum(adjL, axis=-1, keepdims=True)
        deg += jnp.sum(adjR, axis=-1, keepdims=True)
        deg = jnp.clip(deg, 1e-6, None)
        agg = _mm(adjL, a_ref[k, :_HALF]) + _mm(adjR, a_ref[k, _HALF:])
        agg = agg / deg
        emb = jnp.maximum(_mm(agg, wagg_ref[k]) + bagg_ref[k][None, :], 0.0)
        acc = _mm(emb, wfcra_ref[k, :_EMB]) + _mm(h, wfcra_ref[k, _EMB:])
        h = jnp.maximum(acc + bfcra_ref[k][None, :], 0.0)
    out_ref[...] = h


@jax.jit
def kernel(h0, a, adjacent_mat, W_agg, b_agg, W_fcra, b_fcra):
    grid = (_NUM_AGENT // _M,)
    return pl.pallas_call(
        _dhgn_block,
        grid=grid,
        in_specs=[
            pl.BlockSpec((_M, _EMB), lambda i: (i, 0)),
            pl.BlockSpec((_DEPTH, _NUM_AGENT, _IN), lambda i: (0, 0, 0)),
            pl.BlockSpec((_DEPTH, _M, _HALF), lambda i: (0, i, 0)),
            pl.BlockSpec((_DEPTH, _M, _HALF), lambda i: (0, i, 1)),
            pl.BlockSpec((_DEPTH, _IN, _EMB), lambda i: (0, 0, 0)),
            pl.BlockSpec((_DEPTH, _EMB), lambda i: (0, 0)),
            pl.BlockSpec((_DEPTH, _IN, _EMB), lambda i: (0, 0, 0)),
            pl.BlockSpec((_DEPTH, _EMB), lambda i: (0, 0)),
        ],
        out_specs=pl.BlockSpec((_M, _EMB), lambda i: (i, 0)),
        out_shape=jax.ShapeDtypeStruct((_NUM_AGENT, _EMB), jnp.float32),
    )(h0, a, adjacent_mat, adjacent_mat, W_agg, b_agg, W_fcra, b_fcra)


# speedup vs baseline: 1.0905x; 1.0037x over previous
import jax
import jax.numpy as jnp
from jax.experimental import pallas as pl

_NUM_AGENT = 2048
_EMB = 256
_IN = 2 * _EMB
_DEPTH = 2
_M = 512


def _dhgn_block(h0_ref, a_ref, adj_ref, wagg_ref, bagg_ref, wfcra_ref,
                bfcra_ref, out_ref):
    h = h0_ref[...]
    for k in range(_DEPTH):
        adj = adj_ref[k]
        deg = jnp.clip(jnp.sum(adj, axis=-1, keepdims=True), 1e-6, None)
        agg = jax.lax.dot_general(
            adj, a_ref[k],
            dimension_numbers=(((1,), (0,)), ((), ())),
            preferred_element_type=jnp.float32)
        agg = agg / deg
        emb = jax.lax.dot_general(
            agg, wagg_ref[k],
            dimension_numbers=(((1,), (0,)), ((), ())),
            preferred_element_type=jnp.float32)
        emb = jnp.maximum(emb + bagg_ref[k][None, :], 0.0)
        acc = jax.lax.dot_general(
            emb, wfcra_ref[k, :_EMB],
            dimension_numbers=(((1,), (0,)), ((), ())),
            preferred_element_type=jnp.float32)
        acc += jax.lax.dot_general(
            h, wfcra_ref[k, _EMB:],
            dimension_numbers=(((1,), (0,)), ((), ())),
            preferred_element_type=jnp.float32)
        h = jnp.maximum(acc + bfcra_ref[k][None, :], 0.0)
    out_ref[...] = h


@jax.jit
def kernel(h0, a, adjacent_mat, W_agg, b_agg, W_fcra, b_fcra):
    grid = (_NUM_AGENT // _M,)
    return pl.pallas_call(
        _dhgn_block,
        grid=grid,
        in_specs=[
            pl.BlockSpec((_M, _EMB), lambda i: (i, 0)),
            pl.BlockSpec((_DEPTH, _NUM_AGENT, _IN), lambda i: (0, 0, 0)),
            pl.BlockSpec((_DEPTH, _M, _NUM_AGENT), lambda i: (0, i, 0)),
            pl.BlockSpec((_DEPTH, _IN, _EMB), lambda i: (0, 0, 0)),
            pl.BlockSpec((_DEPTH, _EMB), lambda i: (0, 0)),
            pl.BlockSpec((_DEPTH, _IN, _EMB), lambda i: (0, 0, 0)),
            pl.BlockSpec((_DEPTH, _EMB), lambda i: (0, 0)),
        ],
        out_specs=pl.BlockSpec((_M, _EMB), lambda i: (i, 0)),
        out_shape=jax.ShapeDtypeStruct((_NUM_AGENT, _EMB), jnp.float32),
    )(h0, a, adjacent_mat, W_agg, b_agg, W_fcra, b_fcra)


# speedup vs baseline: 1.0959x; 1.0050x over previous
import jax
import jax.numpy as jnp
from jax.experimental import pallas as pl
from jax.experimental.pallas import tpu as pltpu

_NUM_AGENT = 2048
_EMB = 256
_IN = 2 * _EMB
_DEPTH = 2
_M = 512


def _dhgn_block(h0_ref, a_ref, adj_ref, wagg_ref, bagg_ref, wfcra_ref,
                bfcra_ref, out_ref):
    h = h0_ref[...]
    for k in range(_DEPTH):
        adj = adj_ref[k]
        deg = jnp.clip(jnp.sum(adj, axis=-1, keepdims=True), 1e-6, None)
        agg = jax.lax.dot_general(
            adj, a_ref[k],
            dimension_numbers=(((1,), (0,)), ((), ())),
            preferred_element_type=jnp.float32)
        agg = agg / deg
        emb = jax.lax.dot_general(
            agg, wagg_ref[k],
            dimension_numbers=(((1,), (0,)), ((), ())),
            preferred_element_type=jnp.float32)
        emb = jnp.maximum(emb + bagg_ref[k][None, :], 0.0)
        acc = jax.lax.dot_general(
            emb, wfcra_ref[k, :_EMB],
            dimension_numbers=(((1,), (0,)), ((), ())),
            preferred_element_type=jnp.float32)
        acc += jax.lax.dot_general(
            h, wfcra_ref[k, _EMB:],
            dimension_numbers=(((1,), (0,)), ((), ())),
            preferred_element_type=jnp.float32)
        h = jnp.maximum(acc + bfcra_ref[k][None, :], 0.0)
    out_ref[...] = h


@jax.jit
def kernel(h0, a, adjacent_mat, W_agg, b_agg, W_fcra, b_fcra):
    grid = (_NUM_AGENT // _M,)
    return pl.pallas_call(
        _dhgn_block,
        grid=grid,
        in_specs=[
            pl.BlockSpec((_M, _EMB), lambda i: (i, 0)),
            pl.BlockSpec((_DEPTH, _NUM_AGENT, _IN), lambda i: (0, 0, 0)),
            pl.BlockSpec((_DEPTH, _M, _NUM_AGENT), lambda i: (0, i, 0)),
            pl.BlockSpec((_DEPTH, _IN, _EMB), lambda i: (0, 0, 0)),
            pl.BlockSpec((_DEPTH, _EMB), lambda i: (0, 0)),
            pl.BlockSpec((_DEPTH, _IN, _EMB), lambda i: (0, 0, 0)),
            pl.BlockSpec((_DEPTH, _EMB), lambda i: (0, 0)),
        ],
        out_specs=pl.BlockSpec((_M, _EMB), lambda i: (i, 0)),
        out_shape=jax.ShapeDtypeStruct((_NUM_AGENT, _EMB), jnp.float32),
        compiler_params=pltpu.CompilerParams(
            dimension_semantics=("parallel",)),
    )(h0, a, adjacent_mat, W_agg, b_agg, W_fcra, b_fcra)
